# baseline (speedup 1.0000x reference)
import functools

import numpy as np
import jax
import jax.numpy as jnp
from jax import lax
from jax.experimental import pallas as pl
from jax.experimental.pallas import tpu as pltpu
from jax.experimental.pallas import tpu_sc as plsc


_thr_cache = {}


def _threefry2x32_np(k0, k1, x0, x1):
    def rotl(v, r):
        return ((v << np.uint32(r)) | (v >> np.uint32(32 - r))).astype(np.uint32)
    ks = [np.uint32(k0), np.uint32(k1),
          np.uint32(np.uint32(k0) ^ np.uint32(k1) ^ np.uint32(0x1BD11BDA))]
    x = [(x0 + ks[0]).astype(np.uint32), (x1 + ks[1]).astype(np.uint32)]
    rotations = [[13, 15, 26, 6], [17, 29, 16, 24]]
    for i in range(5):
        for r in rotations[i % 2]:
            x[0] = (x[0] + x[1]).astype(np.uint32)
            x[1] = rotl(x[1], r)
            x[1] = x[0] ^ x[1]
        x[0] = (x[0] + ks[(i + 1) % 3]).astype(np.uint32)
        x[1] = (x[1] + ks[(i + 2) % 3] + np.uint32(i + 1)).astype(np.uint32)
    return x


def _gumbel_threshold(E: int) -> np.ndarray:
    if E not in _thr_cache:
        size = E * 2
        i64 = np.arange(size, dtype=np.uint64)
        hi = (i64 >> np.uint64(32)).astype(np.uint32)
        lo = (i64 & np.uint64(0xFFFFFFFF)).astype(np.uint32)
        o0, o1 = _threefry2x32_np(0, 12345, hi, lo)
        bits = (o0 ^ o1).reshape(E, 2)
        f = ((bits >> np.uint32(9)) | np.uint32(0x3F800000)).view(np.float32)
        m = (f - np.float32(1.0)).astype(np.float32)
        scale = np.float32(np.float32(1.0) - np.float32(1e-20))
        u = np.maximum(np.float32(1e-20),
                       (m * scale + np.float32(1e-20)).astype(np.float32))
        g = (-np.log(-np.log(u))).astype(np.float32)
        _thr_cache[E] = np.asarray(g[:, 1] - g[:, 0], dtype=np.float32)
    return _thr_cache[E]


@functools.lru_cache(maxsize=None)
def _make_sc_kernel(N: int, D: int, E: int):
    info = plsc.get_sparse_core_info()
    NC, NS, L = info.num_cores, info.num_subcores, info.num_lanes
    NW = NC * NS
    assert E % NW == 0
    B = E // NW
    C = 80
    assert B % C == 0 and C % L == 0
    G = B // C

    mesh = plsc.VectorSubcoreMesh(core_axis_name="c", subcore_axis_name="s")

    assert G % 2 == 1 and G >= 3
    scratch = [
        pltpu.VMEM((N,), jnp.float32),
        pltpu.VMEM((N,), jnp.float32),
        pltpu.VMEM((B,), jnp.int32),
        pltpu.VMEM((B,), jnp.int32),
        pltpu.VMEM((B,), jnp.float32),
        pltpu.VMEM((C, D), jnp.float32),
        pltpu.VMEM((C, D), jnp.float32),
        pltpu.VMEM((C, D), jnp.float32),
        pltpu.VMEM((C, D), jnp.float32),
        pltpu.VMEM((B,), jnp.float32),
        pltpu.VMEM((B,), jnp.float32),
        pltpu.VMEM((L, L + 1), jnp.float32),
        pltpu.SemaphoreType.DMA,
        pltpu.SemaphoreType.DMA,
    ]

    @functools.partial(
        pl.kernel,
        mesh=mesh,
        out_type=(jax.ShapeDtypeStruct((E,), jnp.float32),
                  jax.ShapeDtypeStruct((E,), jnp.float32)),
        compiler_params=pltpu.CompilerParams(needs_layout_passes=False),
        scratch_types=scratch,
    )
    def k(z1_hbm, col0_hbm, col1_hbm, src_hbm, dst_hbm, thr_hbm,
          out_hbm, a_hbm,
          col0_v, col1_v, sidx_v, didx_v, thr_v, rs0, rd0, rs1, rd1,
          out_v, a_v, trans_v, sem0, sem1):
        wid = lax.axis_index("s") * NC + lax.axis_index("c")
        base = wid * B
        pltpu.sync_copy(col0_hbm.at[pl.ds(0, N)], col0_v)
        pltpu.sync_copy(col1_hbm.at[pl.ds(0, N)], col1_v)
        pltpu.sync_copy(src_hbm.at[pl.ds(base, B)], sidx_v)
        pltpu.sync_copy(dst_hbm.at[pl.ds(base, B)], didx_v)
        pltpu.sync_copy(thr_hbm.at[pl.ds(base, B)], thr_v)

        bufs = ((rs0, rd0, sem0), (rs1, rd1, sem1))

        def issue(g, buf):
            rows_s, rows_d, sem = buf
            off = g * C
            pltpu.async_copy(z1_hbm.at[sidx_v.at[pl.ds(off, C)]],
                             rows_s, sem)
            pltpu.async_copy(z1_hbm.at[didx_v.at[pl.ds(off, C)]],
                             rows_d, sem)

        def wait(buf):
            rows_s, rows_d, sem = buf
            pltpu.make_async_copy(z1_hbm.at[sidx_v.at[pl.ds(0, C)]],
                                  rows_s, sem).wait()
            pltpu.make_async_copy(z1_hbm.at[didx_v.at[pl.ds(0, C)]],
                                  rows_d, sem).wait()

        def compute(g, buf):
            rows_s, rows_d, _ = buf
            off = g * C
            lane = lax.iota(jnp.int32, L)

            def t_body(t, _):
                for i in range(L):
                    e = t * L + i
                    s0 = rows_s[e, pl.ds(0, L)]
                    d0 = rows_d[e, pl.ds(0, L)]
                    acc = s0 * d0
                    for j in range(1, D // L):
                        acc = acc + (rows_s[e, pl.ds(j * L, L)]
                                     * rows_d[e, pl.ds(j * L, L)])
                    trans_v[i, pl.ds(0, L)] = acc
                cols = [plsc.load_gather(trans_v,
                                         [lane, jnp.full((L,), c, jnp.int32)])
                        for c in range(L)]
                while len(cols) > 1:
                    cols = [cols[i] + cols[i + 1]
                            for i in range(0, len(cols), 2)]
                sidx16 = sidx_v[pl.ds(off + t * L, L)]
                didx16 = didx_v[pl.ds(off + t * L, L)]
                c0s = plsc.load_gather(col0_v, [sidx16])
                c0d = plsc.load_gather(col0_v, [didx16])
                c1s = plsc.load_gather(col1_v, [sidx16])
                c1d = plsc.load_gather(col1_v, [didx16])
                res_f = cols[0] - c0s * c0d - c1s * c1d
                res_n = c0s + c0d
                out_v[pl.ds(off + t * L, L)] = res_f
                a_v[pl.ds(off + t * L, L)] = res_n
                return ()

            lax.fori_loop(0, C // L, t_body, (), unroll=False)

        issue(0, bufs[0])

        def pair_body(g2, _):
            c0 = 2 * g2
            issue(c0 + 1, bufs[1])
            wait(bufs[0])
            compute(c0, bufs[0])
            issue(c0 + 2, bufs[0])
            wait(bufs[1])
            compute(c0 + 1, bufs[1])
            return ()

        lax.fori_loop(0, (G - 1) // 2, pair_body, (), unroll=False)
        wait(bufs[0])
        compute(G - 1, bufs[0])

        def fin_body(q, _):
            for u in range(5):
                o = (q * 5 + u) * L
                rf = out_v[pl.ds(o, L)]
                rn = a_v[pl.ds(o, L)]
                th = thr_v[pl.ds(o, L)]
                m = rf >= th
                sf = 1.0 / (1.0 + jnp.exp(-rf))
                sn = 1.0 / (1.0 + jnp.exp(-rn))
                out_v[pl.ds(o, L)] = jnp.where(m, sf, sn)
                a_v[pl.ds(o, L)] = jnp.where(
                    m, jnp.float32(1.0), jnp.float32(0.0))
            return ()

        lax.fori_loop(0, B // (5 * L), fin_body, (), unroll=False)
        pltpu.sync_copy(out_v, out_hbm.at[pl.ds(base, B)])
        pltpu.sync_copy(a_v, a_hbm.at[pl.ds(base, B)])

    return k


def kernel(z1, temp, edge_index):
    del temp
    N, D = z1.shape
    E = edge_index.shape[1]
    src = edge_index[0]
    dst = edge_index[1]
    thr = jnp.asarray(_gumbel_threshold(E))
    out, a = _make_sc_kernel(N, D, E)(z1, z1[:, 0], z1[:, 1], src, dst, thr)
    return out, a[:, None]


# speedup vs baseline: 1.0279x; 1.0279x over previous
import functools

import numpy as np
import jax
import jax.numpy as jnp
from jax import lax
from jax.experimental import pallas as pl
from jax.experimental.pallas import tpu as pltpu
from jax.experimental.pallas import tpu_sc as plsc


_thr_cache = {}


def _threefry2x32_np(k0, k1, x0, x1):
    def rotl(v, r):
        return ((v << np.uint32(r)) | (v >> np.uint32(32 - r))).astype(np.uint32)
    ks = [np.uint32(k0), np.uint32(k1),
          np.uint32(np.uint32(k0) ^ np.uint32(k1) ^ np.uint32(0x1BD11BDA))]
    x = [(x0 + ks[0]).astype(np.uint32), (x1 + ks[1]).astype(np.uint32)]
    rotations = [[13, 15, 26, 6], [17, 29, 16, 24]]
    for i in range(5):
        for r in rotations[i % 2]:
            x[0] = (x[0] + x[1]).astype(np.uint32)
            x[1] = rotl(x[1], r)
            x[1] = x[0] ^ x[1]
        x[0] = (x[0] + ks[(i + 1) % 3]).astype(np.uint32)
        x[1] = (x[1] + ks[(i + 2) % 3] + np.uint32(i + 1)).astype(np.uint32)
    return x


def _gumbel_threshold(E: int) -> np.ndarray:
    if E not in _thr_cache:
        size = E * 2
        i64 = np.arange(size, dtype=np.uint64)
        hi = (i64 >> np.uint64(32)).astype(np.uint32)
        lo = (i64 & np.uint64(0xFFFFFFFF)).astype(np.uint32)
        o0, o1 = _threefry2x32_np(0, 12345, hi, lo)
        bits = (o0 ^ o1).reshape(E, 2)
        f = ((bits >> np.uint32(9)) | np.uint32(0x3F800000)).view(np.float32)
        m = (f - np.float32(1.0)).astype(np.float32)
        scale = np.float32(np.float32(1.0) - np.float32(1e-20))
        u = np.maximum(np.float32(1e-20),
                       (m * scale + np.float32(1e-20)).astype(np.float32))
        g = (-np.log(-np.log(u))).astype(np.float32)
        _thr_cache[E] = np.asarray(g[:, 1] - g[:, 0], dtype=np.float32)
    return _thr_cache[E]


@functools.lru_cache(maxsize=None)
def _make_sc_kernel(N: int, D: int, E: int):
    info = plsc.get_sparse_core_info()
    NC, NS, L = info.num_cores, info.num_subcores, info.num_lanes
    NW = NC * NS
    assert E % NW == 0
    B = E // NW
    C = 80
    assert B % C == 0 and C % L == 0
    G = B // C

    mesh = plsc.VectorSubcoreMesh(core_axis_name="c", subcore_axis_name="s")

    assert G % 2 == 1 and G >= 3
    scratch = [
        pltpu.VMEM((N,), jnp.float32),
        pltpu.VMEM((B,), jnp.int32),
        pltpu.VMEM((B,), jnp.int32),
        pltpu.VMEM((B,), jnp.float32),
        pltpu.VMEM((C, D), jnp.float32),
        pltpu.VMEM((C, D), jnp.float32),
        pltpu.VMEM((C, D), jnp.float32),
        pltpu.VMEM((C, D), jnp.float32),
        pltpu.VMEM((B,), jnp.float32),
        pltpu.VMEM((B,), jnp.float32),
        pltpu.VMEM((L, L + 1), jnp.float32),
        pltpu.SemaphoreType.DMA,
        pltpu.SemaphoreType.DMA,
    ]

    @functools.partial(
        pl.kernel,
        mesh=mesh,
        out_type=(jax.ShapeDtypeStruct((E,), jnp.float32),
                  jax.ShapeDtypeStruct((E,), jnp.float32)),
        compiler_params=pltpu.CompilerParams(needs_layout_passes=False),
        scratch_types=scratch,
    )
    def k(z1_hbm, col0_hbm, src_hbm, dst_hbm, thr_hbm, out_hbm, a_hbm,
          col0_v, sidx_v, didx_v, thr_v, rs0, rd0, rs1, rd1,
          out_v, a_v, trans_v, sem0, sem1):
        wid = lax.axis_index("s") * NC + lax.axis_index("c")
        base = wid * B
        pltpu.sync_copy(col0_hbm.at[pl.ds(0, N)], col0_v)
        pltpu.sync_copy(src_hbm.at[pl.ds(base, B)], sidx_v)
        pltpu.sync_copy(dst_hbm.at[pl.ds(base, B)], didx_v)
        pltpu.sync_copy(thr_hbm.at[pl.ds(base, B)], thr_v)

        bufs = ((rs0, rd0, sem0), (rs1, rd1, sem1))

        def issue(g, buf):
            rows_s, rows_d, sem = buf
            off = g * C
            pltpu.async_copy(z1_hbm.at[sidx_v.at[pl.ds(off, C)]],
                             rows_s, sem)
            pltpu.async_copy(z1_hbm.at[didx_v.at[pl.ds(off, C)]],
                             rows_d, sem)

        def wait(buf):
            rows_s, rows_d, sem = buf
            pltpu.make_async_copy(z1_hbm.at[sidx_v.at[pl.ds(0, C)]],
                                  rows_s, sem).wait()
            pltpu.make_async_copy(z1_hbm.at[didx_v.at[pl.ds(0, C)]],
                                  rows_d, sem).wait()

        def compute(g, buf):
            rows_s, rows_d, _ = buf
            off = g * C
            lane = lax.iota(jnp.int32, L)

            def t_body(t, _):
                for i in range(L):
                    e = t * L + i
                    s0 = rows_s[e, pl.ds(0, L)]
                    d0 = rows_d[e, pl.ds(0, L)]
                    acc = jnp.where(lane >= 2, s0 * d0, jnp.float32(0.0))
                    for j in range(1, D // L):
                        acc = acc + (rows_s[e, pl.ds(j * L, L)]
                                     * rows_d[e, pl.ds(j * L, L)])
                    trans_v[i, pl.ds(0, L)] = acc
                cols = [plsc.load_gather(trans_v,
                                         [lane, jnp.full((L,), c, jnp.int32)])
                        for c in range(L)]
                while len(cols) > 1:
                    cols = [cols[i] + cols[i + 1]
                            for i in range(0, len(cols), 2)]
                res_f = cols[0]
                sidx16 = sidx_v[pl.ds(off + t * L, L)]
                didx16 = didx_v[pl.ds(off + t * L, L)]
                res_n = (plsc.load_gather(col0_v, [sidx16])
                         + plsc.load_gather(col0_v, [didx16]))
                out_v[pl.ds(off + t * L, L)] = res_f
                a_v[pl.ds(off + t * L, L)] = res_n
                return ()

            lax.fori_loop(0, C // L, t_body, (), unroll=False)

        issue(0, bufs[0])

        def pair_body(g2, _):
            c0 = 2 * g2
            issue(c0 + 1, bufs[1])
            wait(bufs[0])
            compute(c0, bufs[0])
            issue(c0 + 2, bufs[0])
            wait(bufs[1])
            compute(c0 + 1, bufs[1])
            return ()

        lax.fori_loop(0, (G - 1) // 2, pair_body, (), unroll=False)
        wait(bufs[0])
        compute(G - 1, bufs[0])

        def fin_body(q, _):
            for u in range(5):
                o = (q * 5 + u) * L
                rf = out_v[pl.ds(o, L)]
                rn = a_v[pl.ds(o, L)]
                th = thr_v[pl.ds(o, L)]
                m = rf >= th
                sf = 1.0 / (1.0 + jnp.exp(-rf))
                sn = 1.0 / (1.0 + jnp.exp(-rn))
                out_v[pl.ds(o, L)] = jnp.where(m, sf, sn)
                a_v[pl.ds(o, L)] = jnp.where(
                    m, jnp.float32(1.0), jnp.float32(0.0))
            return ()

        lax.fori_loop(0, B // (5 * L), fin_body, (), unroll=False)
        pltpu.sync_copy(out_v, out_hbm.at[pl.ds(base, B)])
        pltpu.sync_copy(a_v, a_hbm.at[pl.ds(base, B)])

    return k


def kernel(z1, temp, edge_index):
    del temp
    N, D = z1.shape
    E = edge_index.shape[1]
    src = edge_index[0]
    dst = edge_index[1]
    thr = jnp.asarray(_gumbel_threshold(E))
    out, a = _make_sc_kernel(N, D, E)(z1, z1[:, 0], src, dst, thr)
    return out, a[:, None]
